# baseline (speedup 1.0000x reference)
import functools

import jax
import jax.numpy as jnp
from jax import lax
from jax.experimental import pallas as pl
from jax.experimental.pallas import tpu as pltpu
from jax.experimental.pallas import tpu_sc as plsc

NSTREAM = 4


@functools.cache
def _make(n_cols: int, n_tokens: int, dim: int):
    info = plsc.get_sparse_core_info()
    nw = info.num_cores * info.num_subcores
    tpw = n_tokens // nw
    seg = tpw // NSTREAM
    npair = n_cols // 2
    mesh = plsc.VectorSubcoreMesh(core_axis_name="c", subcore_axis_name="s")
    lanes = info.num_lanes
    tpad = tpw + 1

    @functools.partial(
        pl.kernel,
        mesh=mesh,
        out_type=jax.ShapeDtypeStruct((n_cols, dim, n_tokens), jnp.float32),
        scratch_types=[
            pltpu.VMEM((n_cols, tpw), jnp.int32),
            pltpu.VMEM((tpw, dim), jnp.float32),
            pltpu.VMEM((tpw, dim), jnp.float32),
            pltpu.VMEM((dim, tpad), jnp.float32),
            pltpu.VMEM((dim, tpad), jnp.float32),
            pltpu.SemaphoreType.DMA,
            pltpu.SemaphoreType.DMA,
            pltpu.SemaphoreType.DMA,
            pltpu.SemaphoreType.DMA,
        ],
        compiler_params=pltpu.CompilerParams(
            use_tc_tiling_on_sc=False, needs_layout_passes=False
        ),
    )
    def emb(xt_hbm, table_hbm, out_hbm, idx_t, rows0, rows1, tb0, tb1,
            gsem0, gsem1, wsem0, wsem1):
        wid = lax.axis_index("s") * info.num_cores + lax.axis_index("c")
        i0 = wid * tpw
        pltpu.sync_copy(xt_hbm.at[:, pl.ds(i0, tpw)], idx_t)
        rows = (rows0, rows1)
        tbufs = (tb0, tb1)
        gsems = (gsem0, gsem1)
        wsems = (wsem0, wsem1)

        def gather_descs(j, p):
            return [
                pltpu.make_async_copy(
                    table_hbm.at[idx_t.at[j, pl.ds(q * seg, seg)]],
                    rows[p].at[pl.ds(q * seg, seg)],
                    gsems[p],
                )
                for q in range(NSTREAM)
            ]

        def wb_desc(j, p):
            return pltpu.make_async_copy(
                tbufs[p].at[:, pl.ds(0, tpw)],
                out_hbm.at[j, :, pl.ds(i0, tpw)],
                wsems[p],
            )

        dlo = lax.iota(jnp.int32, lanes)
        dhi = dlo + lanes

        def transpose(p):
            src, dst = rows[p], tbufs[p]

            @plsc.parallel_loop(0, tpw, 8, unroll=2)
            def tbody(i):
                for t in range(8):
                    iv = jnp.full((lanes,), i + t, jnp.int32)
                    v0 = src[i + t, pl.ds(0, lanes)]
                    v1 = src[i + t, pl.ds(lanes, lanes)]
                    plsc.store_scatter(dst, [dlo, iv], v0)
                    plsc.store_scatter(dst, [dhi, iv], v1)

        for d in gather_descs(0, 0):
            d.start()
        for d in gather_descs(1, 1):
            d.start()

        def body(jj, carry):
            for p in range(2):
                j = 2 * jj + p
                for desc in gather_descs(j, p):
                    desc.wait()

                @pl.when(jj > 0)
                def _():
                    wb_desc(j - 2, p).wait()

                transpose(p)
                wb_desc(j, p).start()

                @pl.when(jj + 1 < npair)
                def _():
                    for desc in gather_descs(j + 2, p):
                        desc.start()

            return carry

        lax.fori_loop(0, npair, body, 0)
        wb_desc(n_cols - 2, 0).wait()
        wb_desc(n_cols - 1, 1).wait()

    return emb


@functools.cache
def _make_tc_relayout(v: int, dim: int):
    bc = 512
    bo = bc * dim // 128
    grid = (v + bc - 1) // bc
    nsub = 128 // dim

    def body(wt_ref, out_ref, p_ref):
        @pl.when(pl.program_id(0) == 0)
        def _():
            r = lax.broadcasted_iota(jnp.int32, (bc, bc), 0)
            k = lax.broadcasted_iota(jnp.int32, (bc, bc), 1)
            p_ref[...] = (k == nsub * (r % bo) + r // bo).astype(jnp.float32)

        x = wt_ref[...]
        t = jax.lax.dot_general(
            p_ref[...], x, (((1,), (1,)), ((), ())),
            preferred_element_type=jnp.float32,
            precision=jax.lax.Precision.HIGHEST,
        )
        out_ref[...] = jnp.concatenate(
            [lax.slice(t, (s * bo, 0), ((s + 1) * bo, dim)) for s in range(nsub)],
            axis=1,
        )

    return pl.pallas_call(
        body,
        grid=(grid,),
        in_specs=[pl.BlockSpec((dim, bc), lambda i: (0, i))],
        out_specs=pl.BlockSpec((bo, 128), lambda i: (i, 0)),
        out_shape=jax.ShapeDtypeStruct((v * dim // 128, 128), jnp.float32),
        scratch_shapes=[pltpu.VMEM((bc, bc), jnp.float32)],
    )


def kernel(x, weights):
    b, s = x.shape
    v, dim = weights.shape
    xt = jnp.transpose(x.astype(jnp.int32))
    wt = jnp.transpose(weights)
    table = _make_tc_relayout(v, dim)(wt).reshape(v, dim)
    out_t = _make(s, b, dim)(xt, table)
    return jnp.transpose(out_t, (2, 0, 1))


# speedup vs baseline: 3.3429x; 3.3429x over previous
import functools

import jax
import jax.numpy as jnp
from jax import lax
from jax.experimental import pallas as pl
from jax.experimental.pallas import tpu as pltpu
from jax.experimental.pallas import tpu_sc as plsc

NSTREAM = 4


@functools.cache
def _make(n_cols: int, n_tokens: int, dim: int):
    info = plsc.get_sparse_core_info()
    nw = info.num_cores * info.num_subcores
    tpw = n_tokens // nw
    seg = tpw // NSTREAM
    npair = n_cols // 2
    mesh = plsc.VectorSubcoreMesh(core_axis_name="c", subcore_axis_name="s")
    lanes = info.num_lanes
    tpad = tpw + 1

    @functools.partial(
        pl.kernel,
        mesh=mesh,
        out_type=jax.ShapeDtypeStruct((n_cols, dim, n_tokens), jnp.float32),
        scratch_types=[
            pltpu.VMEM((n_cols, tpw), jnp.int32),
            pltpu.VMEM((tpw, dim), jnp.float32),
            pltpu.VMEM((tpw, dim), jnp.float32),
            pltpu.VMEM((dim, tpad), jnp.float32),
            pltpu.VMEM((dim, tpad), jnp.float32),
            pltpu.SemaphoreType.DMA,
            pltpu.SemaphoreType.DMA,
            pltpu.SemaphoreType.DMA,
            pltpu.SemaphoreType.DMA,
        ],
        compiler_params=pltpu.CompilerParams(
            use_tc_tiling_on_sc=False, needs_layout_passes=False
        ),
    )
    def emb(xt_hbm, table_hbm, out_hbm, idx_t, rows0, rows1, tb0, tb1,
            gsem0, gsem1, wsem0, wsem1):
        wid = lax.axis_index("s") * info.num_cores + lax.axis_index("c")
        i0 = wid * tpw
        pltpu.sync_copy(xt_hbm.at[:, pl.ds(i0, tpw)], idx_t)
        rows = (rows0, rows1)
        tbufs = (tb0, tb1)
        gsems = (gsem0, gsem1)
        wsems = (wsem0, wsem1)

        def gather_descs(j, p):
            return [
                pltpu.make_async_copy(
                    table_hbm.at[idx_t.at[j, pl.ds(q * seg, seg)]],
                    rows[p].at[pl.ds(q * seg, seg)],
                    gsems[p],
                )
                for q in range(NSTREAM)
            ]

        def wb_desc(j, p):
            return pltpu.make_async_copy(
                tbufs[p].at[:, pl.ds(0, tpw)],
                out_hbm.at[j, :, pl.ds(i0, tpw)],
                wsems[p],
            )

        dlo = lax.iota(jnp.int32, lanes)
        dhi = dlo + lanes

        def transpose(p):
            src, dst = rows[p], tbufs[p]

            @plsc.parallel_loop(0, tpw, 8, unroll=2)
            def tbody(i):
                for t in range(8):
                    iv = jnp.full((lanes,), i + t, jnp.int32)
                    v0 = src[i + t, pl.ds(0, lanes)]
                    v1 = src[i + t, pl.ds(lanes, lanes)]
                    plsc.store_scatter(dst, [dlo, iv], v0)
                    plsc.store_scatter(dst, [dhi, iv], v1)

        for d in gather_descs(0, 0):
            d.start()
        for d in gather_descs(1, 1):
            d.start()

        def body(jj, carry):
            for p in range(2):
                j = 2 * jj + p
                for desc in gather_descs(j, p):
                    desc.wait()

                @pl.when(jj > 0)
                def _():
                    wb_desc(j - 2, p).wait()

                transpose(p)
                wb_desc(j, p).start()

                @pl.when(jj + 1 < npair)
                def _():
                    for desc in gather_descs(j + 2, p):
                        desc.start()

            return carry

        lax.fori_loop(0, npair, body, 0)
        wb_desc(n_cols - 2, 0).wait()
        wb_desc(n_cols - 1, 1).wait()

    return emb


def kernel(x, weights):
    b, s = x.shape
    dim = weights.shape[1]
    xt = jnp.transpose(x.astype(jnp.int32))
    out_t = _make(s, b, dim)(xt, weights)
    return jnp.transpose(out_t, (2, 0, 1))
